# baseline (speedup 1.0000x reference)
import functools

import jax
import jax.numpy as jnp
from jax import lax
from jax.experimental import pallas as pl
from jax.experimental.pallas import tpu as pltpu
from jax.experimental.pallas import tpu_sc as plsc

N_NODES = 10000
PAD_N = 10240
D = 128
N_EDGES = 320000
NW = 32
CHUNK = 128
CH = 80
GROUP = 8
NG = CH // GROUP
CH_TOT = (NG + 2) * GROUP
ROWS_PER_TILE = PAD_N // 16


def _zero16():
    return jnp.zeros((16,), jnp.float32)


def _deg_body(dst_hbm, deg_out, deg_spmem, idxbuf, ones, stage, sem):
    cid = lax.axis_index("c")
    sid = lax.axis_index("s")
    wid = cid * 16 + sid
    base = sid * ROWS_PER_TILE

    for j in range(CHUNK // 16):
        ones[pl.ds(j * 16, 16)] = jnp.full((16,), 1.0, jnp.float32)

    def zb(i, _):
        stage[pl.ds(i * 16, 16)] = _zero16()
        return 0
    lax.fori_loop(0, ROWS_PER_TILE // 16, zb, 0)

    pltpu.sync_copy(stage, deg_spmem.at[pl.ds(base, ROWS_PER_TILE)])
    pltpu.async_copy(dst_hbm.at[wid, pl.ds(0, CH)], idxbuf, sem).wait()
    plsc.subcore_barrier()

    def hist(ch, _):
        pltpu.sync_copy(ones, deg_spmem.at[idxbuf.at[ch]], add=True)
        return 0
    lax.fori_loop(0, CH, hist, 0)
    plsc.subcore_barrier()

    pltpu.sync_copy(deg_spmem.at[pl.ds(base, ROWS_PER_TILE)], stage)
    pltpu.sync_copy(stage, deg_out.at[cid, pl.ds(base, ROWS_PER_TILE)])


_deg_kernel = functools.partial(
    pl.kernel,
    out_type=jax.ShapeDtypeStruct((2, PAD_N), jnp.float32),
    mesh=plsc.VectorSubcoreMesh(core_axis_name="c", subcore_axis_name="s"),
    scratch_types=[
        pltpu.VMEM_SHARED((PAD_N,), jnp.float32),
        pltpu.VMEM((CH, CHUNK), jnp.int32),
        pltpu.VMEM((CHUNK,), jnp.float32),
        pltpu.VMEM((ROWS_PER_TILE,), jnp.float32),
        pltpu.SemaphoreType.DMA,
    ],
)(_deg_body)


BLK = 1280


def _hprime_body(x_ref, w_ref, degp_ref, out_ref, dis_ref):
    deg = degp_ref[0, :] + degp_ref[1, :] + 1.0
    dis = lax.rsqrt(deg)
    h = jnp.dot(x_ref[...], w_ref[...], preferred_element_type=jnp.float32)
    out_ref[...] = h * dis[:, None]
    dis_ref[...] = jnp.broadcast_to(dis[:, None], (BLK, 2))


def _hprime(x, w, deg_parts):
    return pl.pallas_call(
        _hprime_body,
        grid=(PAD_N // BLK,),
        in_specs=[
            pl.BlockSpec((BLK, D), lambda i: (i, 0)),
            pl.BlockSpec((D, D), lambda i: (0, 0)),
            pl.BlockSpec((2, BLK), lambda i: (0, i)),
        ],
        out_specs=[
            pl.BlockSpec((BLK, D), lambda i: (i, 0)),
            pl.BlockSpec((BLK, 2), lambda i: (i, 0)),
        ],
        out_shape=[
            jax.ShapeDtypeStruct((PAD_N, D), jnp.float32),
            jax.ShapeDtypeStruct((PAD_N, 2), jnp.float32),
        ],
    )(x, w, deg_parts)


def _scatter_body(hp_hbm, src_hbm, dst_hbm, out_hbm,
                  acc_spmem, sbufa, sbufb, dbufa, dbufb, rows0, rows1,
                  isa, isb, ida, idb, gsem0, gsem1):
    cid = lax.axis_index("c")
    sid = lax.axis_index("s")
    wid = cid * 16 + sid
    base = sid * ROWS_PER_TILE
    rows = (rows0, rows1)
    gsem = (gsem0, gsem1)

    def zb(r, _):
        for j in range(8):
            rows0[r, pl.ds(j * 16, 16)] = _zero16()
        return 0
    lax.fori_loop(0, CHUNK, zb, 0)
    for k in range(ROWS_PER_TILE // CHUNK):
        pltpu.sync_copy(rows0, acc_spmem.at[pl.ds(base + k * CHUNK, CHUNK)])

    def load_group(g, sbuf, ssem, dbuf, dsem):
        pltpu.async_copy(src_hbm.at[wid, pl.ds(g * GROUP, GROUP)], sbuf, ssem)
        pltpu.async_copy(dst_hbm.at[wid, pl.ds(g * GROUP, GROUP)], dbuf, dsem)

    load_group(0, sbufa, isa, dbufa, ida)
    pltpu.make_async_copy(src_hbm.at[wid, pl.ds(0, GROUP)], sbufa, isa).wait()
    pltpu.make_async_copy(dst_hbm.at[wid, pl.ds(0, GROUP)], dbufa, ida).wait()
    load_group(1, sbufb, isb, dbufb, idb)
    pltpu.async_copy(hp_hbm.at[sbufa.at[0]], rows0, gsem0)
    pltpu.async_copy(hp_hbm.at[sbufa.at[1]], rows1, gsem1)
    plsc.subcore_barrier()

    def do_group(sbuf, dbuf, nsbuf, nssem, ndbuf, ndsem):
        for k in range(GROUP):
            r, gs = rows[k % 2], gsem[k % 2]
            pltpu.make_async_copy(hp_hbm.at[sbuf.at[0]], r, gs).wait()
            pltpu.sync_copy(r, acc_spmem.at[dbuf.at[k]], add=True)
            if k < GROUP - 2:
                pltpu.async_copy(hp_hbm.at[sbuf.at[k + 2]], r, gs)
            else:
                if k == GROUP - 2:
                    pltpu.make_async_copy(
                        src_hbm.at[wid, pl.ds(0, GROUP)], nsbuf, nssem).wait()
                else:
                    pltpu.make_async_copy(
                        dst_hbm.at[wid, pl.ds(0, GROUP)], ndbuf, ndsem).wait()
                pltpu.async_copy(hp_hbm.at[nsbuf.at[k - (GROUP - 2)]], r, gs)

    def body(i, _):
        do_group(sbufa, dbufa, sbufb, isb, dbufb, idb)
        load_group(2 * i + 2, sbufa, isa, dbufa, ida)
        do_group(sbufb, dbufb, sbufa, isa, dbufa, ida)
        load_group(2 * i + 3, sbufb, isb, dbufb, idb)
        return 0
    lax.fori_loop(0, NG // 2, body, 0)

    pltpu.make_async_copy(hp_hbm.at[sbufa.at[0]], rows0, gsem0).wait()
    pltpu.make_async_copy(hp_hbm.at[sbufa.at[0]], rows1, gsem1).wait()
    pltpu.make_async_copy(src_hbm.at[wid, pl.ds(0, GROUP)], sbufb, isb).wait()
    pltpu.make_async_copy(dst_hbm.at[wid, pl.ds(0, GROUP)], dbufb, idb).wait()
    plsc.subcore_barrier()

    pltpu.sync_copy(acc_spmem.at[pl.ds(base, ROWS_PER_TILE)],
                    out_hbm.at[cid, pl.ds(base, ROWS_PER_TILE)])


_scatter_kernel = functools.partial(
    pl.kernel,
    out_type=jax.ShapeDtypeStruct((2, PAD_N, D), jnp.float32),
    mesh=plsc.VectorSubcoreMesh(core_axis_name="c", subcore_axis_name="s"),
    scratch_types=[
        pltpu.VMEM_SHARED((PAD_N, D), jnp.float32),
        pltpu.VMEM((GROUP, CHUNK), jnp.int32),
        pltpu.VMEM((GROUP, CHUNK), jnp.int32),
        pltpu.VMEM((GROUP, CHUNK), jnp.int32),
        pltpu.VMEM((GROUP, CHUNK), jnp.int32),
        pltpu.VMEM((CHUNK, D), jnp.float32),
        pltpu.VMEM((CHUNK, D), jnp.float32),
        pltpu.SemaphoreType.DMA,
        pltpu.SemaphoreType.DMA,
        pltpu.SemaphoreType.DMA,
        pltpu.SemaphoreType.DMA,
        pltpu.SemaphoreType.DMA,
        pltpu.SemaphoreType.DMA,
    ],
)(_scatter_body)


def _combine_body(parts_ref, hp_ref, dis_ref, b_ref, out_ref):
    dis = dis_ref[:, 0]
    acc = parts_ref[0] + parts_ref[1] + hp_ref[...]
    out_ref[...] = jnp.maximum(acc * dis[:, None] + b_ref[...], 0.0)


BLK2 = 2000


def _combine(parts, hp, dis2, b2d):
    return pl.pallas_call(
        _combine_body,
        grid=(N_NODES // BLK2,),
        in_specs=[
            pl.BlockSpec((2, BLK2, D), lambda i: (0, i, 0)),
            pl.BlockSpec((BLK2, D), lambda i: (i, 0)),
            pl.BlockSpec((BLK2, 2), lambda i: (i, 0)),
            pl.BlockSpec((1, D), lambda i: (0, 0)),
        ],
        out_specs=pl.BlockSpec((BLK2, D), lambda i: (i, 0)),
        out_shape=jax.ShapeDtypeStruct((N_NODES, D), jnp.float32),
    )(parts, hp, dis2, b2d)


@jax.jit
def kernel(x, edge_index, W, b):
    ei = edge_index.astype(jnp.int32)
    patt = N_NODES + jnp.arange(CHUNK, dtype=jnp.int32)
    n_real_pad = NW * CH * CHUNK - N_EDGES
    pad = jnp.broadcast_to(patt, (n_real_pad // CHUNK, CHUNK)).reshape(-1)
    extra = jnp.broadcast_to(patt, (NW, CH_TOT - CH, CHUNK))

    def paged(idx):
        real = jnp.concatenate([idx, pad]).reshape(NW, CH, CHUNK)
        return jnp.concatenate([real, extra], axis=1)

    src = paged(ei[0])
    dst = paged(ei[1])

    deg_parts = _deg_kernel(dst)
    hp, dis2 = _hprime(x, W, deg_parts)
    parts = _scatter_kernel(hp, src, dst)
    return _combine(parts, hp, dis2, b.reshape(1, D))


# speedup vs baseline: 1.0093x; 1.0093x over previous
import functools

import jax
import jax.numpy as jnp
from jax import lax
from jax.experimental import pallas as pl
from jax.experimental.pallas import tpu as pltpu
from jax.experimental.pallas import tpu_sc as plsc

N_NODES = 10000
PAD_N = 10240
D = 128
N_EDGES = 320000
NW = 32
CHUNK = 128
CH = 80
GROUP = 8
NG = CH // GROUP
CH_TOT = (NG + 2) * GROUP
ROWS_PER_TILE = PAD_N // 16


def _zero16():
    return jnp.zeros((16,), jnp.float32)


def _deg_body(pages_hbm, deg_out, deg_spmem, idxbuf, ones, stage, sem):
    cid = lax.axis_index("c")
    sid = lax.axis_index("s")
    wid = cid * 16 + sid
    base = sid * ROWS_PER_TILE

    for j in range(CHUNK // 16):
        ones[pl.ds(j * 16, 16)] = jnp.full((16,), 1.0, jnp.float32)

    def zb(i, _):
        stage[pl.ds(i * 16, 16)] = _zero16()
        return 0
    lax.fori_loop(0, ROWS_PER_TILE // 16, zb, 0)

    pltpu.sync_copy(stage, deg_spmem.at[pl.ds(base, ROWS_PER_TILE)])
    pltpu.async_copy(pages_hbm.at[wid, pl.ds(0, 2 * CH)], idxbuf, sem).wait()
    plsc.subcore_barrier()

    def hist(ch, _):
        pltpu.sync_copy(ones, deg_spmem.at[idxbuf.at[2 * ch + 1]], add=True)
        return 0
    lax.fori_loop(0, CH, hist, 0)
    plsc.subcore_barrier()

    pltpu.sync_copy(deg_spmem.at[pl.ds(base, ROWS_PER_TILE)], stage)
    pltpu.sync_copy(stage, deg_out.at[cid, pl.ds(base, ROWS_PER_TILE)])


_deg_kernel = functools.partial(
    pl.kernel,
    out_type=jax.ShapeDtypeStruct((2, PAD_N), jnp.float32),
    mesh=plsc.VectorSubcoreMesh(core_axis_name="c", subcore_axis_name="s"),
    scratch_types=[
        pltpu.VMEM_SHARED((PAD_N,), jnp.float32),
        pltpu.VMEM((2 * CH, CHUNK), jnp.int32),
        pltpu.VMEM((CHUNK,), jnp.float32),
        pltpu.VMEM((ROWS_PER_TILE,), jnp.float32),
        pltpu.SemaphoreType.DMA,
    ],
)(_deg_body)


BLK = 1280


def _hprime_body(x_ref, w_ref, degp_ref, out_ref, dis_ref):
    deg = degp_ref[0, :] + degp_ref[1, :] + 1.0
    dis = lax.rsqrt(deg)
    h = jnp.dot(x_ref[...], w_ref[...], preferred_element_type=jnp.float32)
    out_ref[...] = h * dis[:, None]
    dis_ref[...] = jnp.broadcast_to(dis[:, None], (BLK, 2))


def _hprime(x, w, deg_parts):
    return pl.pallas_call(
        _hprime_body,
        grid=(PAD_N // BLK,),
        in_specs=[
            pl.BlockSpec((BLK, D), lambda i: (i, 0)),
            pl.BlockSpec((D, D), lambda i: (0, 0)),
            pl.BlockSpec((2, BLK), lambda i: (0, i)),
        ],
        out_specs=[
            pl.BlockSpec((BLK, D), lambda i: (i, 0)),
            pl.BlockSpec((BLK, 2), lambda i: (i, 0)),
        ],
        out_shape=[
            jax.ShapeDtypeStruct((PAD_N, D), jnp.float32),
            jax.ShapeDtypeStruct((PAD_N, 2), jnp.float32),
        ],
    )(x, w, deg_parts)


def _scatter_body(hp_hbm, pages_hbm, out_hbm,
                  acc_spmem, ibufa, ibufb, rows0, rows1,
                  isa, isb, gsem0, gsem1):
    cid = lax.axis_index("c")
    sid = lax.axis_index("s")
    wid = cid * 16 + sid
    base = sid * ROWS_PER_TILE
    rows = (rows0, rows1)
    gsem = (gsem0, gsem1)
    GR = 2 * GROUP

    def zb(r, _):
        for j in range(8):
            rows0[r, pl.ds(j * 16, 16)] = _zero16()
        return 0
    lax.fori_loop(0, CHUNK, zb, 0)
    for k in range(ROWS_PER_TILE // CHUNK):
        pltpu.sync_copy(rows0, acc_spmem.at[pl.ds(base + k * CHUNK, CHUNK)])

    def load_group(g, buf, sem):
        pltpu.async_copy(pages_hbm.at[wid, pl.ds(g * GR, GR)], buf, sem)

    load_group(0, ibufa, isa)
    pltpu.make_async_copy(pages_hbm.at[wid, pl.ds(0, GR)], ibufa, isa).wait()
    load_group(1, ibufb, isb)
    pltpu.async_copy(hp_hbm.at[ibufa.at[0]], rows0, gsem0)
    pltpu.async_copy(hp_hbm.at[ibufa.at[2]], rows1, gsem1)
    plsc.subcore_barrier()

    def do_group(buf, nbuf, nsem):
        for k in range(GROUP):
            r, gs = rows[k % 2], gsem[k % 2]
            pltpu.make_async_copy(hp_hbm.at[buf.at[0]], r, gs).wait()
            pltpu.sync_copy(r, acc_spmem.at[buf.at[2 * k + 1]], add=True)
            if k < GROUP - 2:
                pltpu.async_copy(hp_hbm.at[buf.at[2 * (k + 2)]], r, gs)
            else:
                if k == GROUP - 2:
                    pltpu.make_async_copy(
                        pages_hbm.at[wid, pl.ds(0, GR)], nbuf, nsem).wait()
                pltpu.async_copy(
                    hp_hbm.at[nbuf.at[2 * (k - (GROUP - 2))]], r, gs)

    def body(i, _):
        do_group(ibufa, ibufb, isb)
        load_group(2 * i + 2, ibufa, isa)
        do_group(ibufb, ibufa, isa)
        load_group(2 * i + 3, ibufb, isb)
        return 0
    lax.fori_loop(0, NG // 2, body, 0)

    pltpu.make_async_copy(hp_hbm.at[ibufa.at[0]], rows0, gsem0).wait()
    pltpu.make_async_copy(hp_hbm.at[ibufa.at[0]], rows1, gsem1).wait()
    pltpu.make_async_copy(pages_hbm.at[wid, pl.ds(0, GR)], ibufb, isb).wait()
    plsc.subcore_barrier()

    pltpu.sync_copy(acc_spmem.at[pl.ds(base, ROWS_PER_TILE)],
                    out_hbm.at[cid, pl.ds(base, ROWS_PER_TILE)])


_scatter_kernel = functools.partial(
    pl.kernel,
    out_type=jax.ShapeDtypeStruct((2, PAD_N, D), jnp.float32),
    mesh=plsc.VectorSubcoreMesh(core_axis_name="c", subcore_axis_name="s"),
    scratch_types=[
        pltpu.VMEM_SHARED((PAD_N, D), jnp.float32),
        pltpu.VMEM((2 * GROUP, CHUNK), jnp.int32),
        pltpu.VMEM((2 * GROUP, CHUNK), jnp.int32),
        pltpu.VMEM((CHUNK, D), jnp.float32),
        pltpu.VMEM((CHUNK, D), jnp.float32),
        pltpu.SemaphoreType.DMA,
        pltpu.SemaphoreType.DMA,
        pltpu.SemaphoreType.DMA,
        pltpu.SemaphoreType.DMA,
    ],
)(_scatter_body)


def _combine_body(parts_ref, hp_ref, dis_ref, b_ref, out_ref):
    dis = dis_ref[:, 0]
    acc = parts_ref[0] + parts_ref[1] + hp_ref[...]
    out_ref[...] = jnp.maximum(acc * dis[:, None] + b_ref[...], 0.0)


BLK2 = 2000


def _combine(parts, hp, dis2, b2d):
    return pl.pallas_call(
        _combine_body,
        grid=(N_NODES // BLK2,),
        in_specs=[
            pl.BlockSpec((2, BLK2, D), lambda i: (0, i, 0)),
            pl.BlockSpec((BLK2, D), lambda i: (i, 0)),
            pl.BlockSpec((BLK2, 2), lambda i: (i, 0)),
            pl.BlockSpec((1, D), lambda i: (0, 0)),
        ],
        out_specs=pl.BlockSpec((BLK2, D), lambda i: (i, 0)),
        out_shape=jax.ShapeDtypeStruct((N_NODES, D), jnp.float32),
    )(parts, hp, dis2, b2d)


@jax.jit
def kernel(x, edge_index, W, b):
    ei = edge_index.astype(jnp.int32)
    pairs = jnp.transpose(ei.reshape(2, N_EDGES // CHUNK, CHUNK),
                          (1, 0, 2)).reshape(2 * N_EDGES // CHUNK, CHUNK)
    patt = N_NODES + jnp.arange(CHUNK, dtype=jnp.int32)
    n_pad_rows = 2 * (NW * CH * CHUNK - N_EDGES) // CHUNK
    pad2d = jnp.broadcast_to(patt, (n_pad_rows, CHUNK))
    extra = jnp.broadcast_to(patt, (NW, 2 * (CH_TOT - CH), CHUNK))
    real = jnp.concatenate([pairs, pad2d]).reshape(NW, 2 * CH, CHUNK)
    pages = jnp.concatenate([real, extra], axis=1)

    deg_parts = _deg_kernel(pages)
    hp, dis2 = _hprime(x, W, deg_parts)
    parts = _scatter_kernel(hp, pages)
    return _combine(parts, hp, dis2, b.reshape(1, D))
